# baseline (speedup 1.0000x reference)
import jax
import jax.numpy as jnp
from jax import lax
from jax.experimental import pallas as pl
from jax.experimental.pallas import tpu as pltpu

N = 8192
DIM = 512
K = 16
BR = 2048
BC = 2048
TL = 128
NT = BC // TL
PAD = 128
RS = 64
UW = 4 * TL
NR = N // BR
NC = N // BC
INF = float("inf")
BIG = 2**30


def _norms_body(x_ref, out_ref):
    x = x_ref[...]
    out_ref[...] = jnp.sum(x * x, axis=1, keepdims=True)


def _cminmax(av, ai, bv, bi):
    c = av <= bv
    return (jnp.where(c, av, bv), jnp.where(c, ai, bi),
            jnp.where(c, bv, av), jnp.where(c, bi, ai))


def _merge11(a, b):
    (av, ai, acnt), (bv, bi, bcnt) = a, b
    lov, loi, hiv, hii = _cminmax(av[0], ai[0], bv[0], bi[0])
    return ([lov, hiv], [loi, hii], acnt + bcnt)


def _merge22(a, b):
    (av, ai, acnt), (bv, bi, bcnt) = a, b
    o0v, o0i, hv, hi = _cminmax(av[0], ai[0], bv[0], bi[0])
    c1 = av[1] <= bv[1]
    m1v = jnp.where(c1, av[1], bv[1])
    m1i = jnp.where(c1, ai[1], bi[1])
    o1v, o1i, o2v, o2i = _cminmax(hv, hi, m1v, m1i)
    return ([o0v, o1v, o2v], [o0i, o1i, o2i], acnt + bcnt)


def _merge33(a, b):
    (av, ai, acnt), (bv, bi, bcnt) = a, b
    o0v, o0i, hv, hi = _cminmax(av[0], ai[0], bv[0], bi[0])
    c1 = av[1] <= bv[1]
    m1v = jnp.where(c1, av[1], bv[1])
    m1i = jnp.where(c1, ai[1], bi[1])
    o1v, o1i, h2v, h2i = _cminmax(hv, hi, m1v, m1i)
    c3 = av[2] <= bv[2]
    m2v = jnp.where(c3, av[2], bv[2])
    m2i = jnp.where(c3, ai[2], bi[2])
    c4 = h2v <= m2v
    o2v = jnp.where(c4, h2v, m2v)
    o2i = jnp.where(c4, h2i, m2i)
    return ([o0v, o1v, o2v], [o0i, o1i, o2i], acnt + bcnt)


def _knn_body(xi_ref, xjt_ref, x2r_ref, x2c_ref, out_ref,
              d_ref, u_ref, ui_ref, cnt_ref, bv_ref, bi_ref,
              nv_ref, ni_ref):
    j = pl.program_id(1)

    @pl.when(j == 0)
    def _init():
        bv_ref[...] = jnp.full((BR, PAD), INF, jnp.float32)
        bi_ref[...] = jnp.full((BR, PAD), BIG, jnp.int32)
        nv_ref[...] = jnp.full((BR, PAD), INF, jnp.float32)
        ni_ref[...] = jnp.full((BR, PAD), BIG, jnp.int32)

    mm = jnp.dot(xi_ref[...], xjt_ref[...],
                 preferred_element_type=jnp.float32)
    d_ref[...] = (x2r_ref[...] + x2c_ref[...]) - 2.0 * mm

    joff = j * BC
    lanes = lax.broadcasted_iota(jnp.int32, (BR, PAD), 1)

    def strip_fold(si, carry):
        r = si * RS
        tv = bv_ref[pl.ds(r, RS), K - 1:K]
        lanes8 = lax.broadcasted_iota(jnp.int32, (RS, TL), 1)
        leaves = []
        for k in range(NT):
            v = d_ref[pl.ds(r, RS), k * TL:(k + 1) * TL]
            keep = v < tv
            fv = jnp.where(keep, v, INF)
            fi = lanes8 + (joff + k * TL)
            leaves.append(([fv], [fi], keep.astype(jnp.int32)))
        leaves = [_merge11(leaves[s], leaves[s + 8]) for s in range(8)]
        leaves = [_merge22(leaves[s], leaves[s + 4]) for s in range(4)]
        leaves = [_merge33(leaves[s], leaves[s + 2]) for s in range(2)]
        s_v, s_i, cnt = _merge33(leaves[0], leaves[1])
        u_ref[pl.ds(r, RS), 0 * TL:1 * TL] = s_v[0]
        u_ref[pl.ds(r, RS), 1 * TL:2 * TL] = s_v[1]
        u_ref[pl.ds(r, RS), 2 * TL:3 * TL] = s_v[2]
        ui_ref[pl.ds(r, RS), 0 * TL:1 * TL] = s_i[0]
        ui_ref[pl.ds(r, RS), 1 * TL:2 * TL] = s_i[1]
        ui_ref[pl.ds(r, RS), 2 * TL:3 * TL] = s_i[2]
        cnt_ref[pl.ds(r, RS), :] = cnt
        return carry

    lax.fori_loop(0, BR // RS, strip_fold, 0)
    overflow = jnp.max(cnt_ref[...]) > 3

    @pl.when(jnp.logical_not(overflow))
    def _fast():
        u_ref[:, 3 * TL:4 * TL] = bv_ref[...]
        ui_ref[:, 3 * TL:4 * TL] = bi_ref[...]

        def pass_u(t, carry):
            uv = u_ref[...]
            uiv = ui_ref[...]
            m = jnp.min(uv, axis=1, keepdims=True)
            am = jnp.min(jnp.where(uv == m, uiv, BIG),
                         axis=1, keepdims=True)
            u_ref[...] = jnp.where(uiv == am, INF, uv)
            nv_ref[...] = jnp.where(lanes == t, m, nv_ref[...])
            ni_ref[...] = jnp.where(lanes == t, am, ni_ref[...])
            return carry

        lax.fori_loop(0, K, pass_u, 0)

    @pl.when(overflow)
    def _slow():
        def pass_t(t, carry):
            dvv = d_ref[...]
            bvv = bv_ref[...]
            biv = bi_ref[...]
            iota = lax.broadcasted_iota(jnp.int32, (BR, BC), 1)
            m = jnp.minimum(jnp.min(dvv, axis=1, keepdims=True),
                            jnp.min(bvv, axis=1, keepdims=True))
            amd = jnp.min(jnp.where(dvv == m, iota, BIG),
                          axis=1, keepdims=True) + joff
            amp = jnp.min(jnp.where(bvv == m, biv, BIG),
                          axis=1, keepdims=True)
            am = jnp.minimum(amp, amd)
            d_ref[...] = jnp.where(iota == (am - joff), INF, dvv)
            bv_ref[...] = jnp.where(biv == am, INF, bvv)
            nv_ref[...] = jnp.where(lanes == t, m, nv_ref[...])
            ni_ref[...] = jnp.where(lanes == t, am, ni_ref[...])
            return carry

        lax.fori_loop(0, K, pass_t, 0)

    bv_ref[...] = nv_ref[...]
    bi_ref[...] = ni_ref[...]
    nv_ref[...] = jnp.full((BR, PAD), INF, jnp.float32)
    ni_ref[...] = jnp.full((BR, PAD), BIG, jnp.int32)

    @pl.when(j == NC - 1)
    def _emit():
        out_ref[...] = bi_ref[:, :K]


def kernel(x):
    x2r = pl.pallas_call(
        _norms_body,
        out_shape=jax.ShapeDtypeStruct((N, 1), jnp.float32),
    )(x)
    xt = x.T
    x2c = x2r.T
    idx = pl.pallas_call(
        _knn_body,
        grid=(NR, NC),
        in_specs=[
            pl.BlockSpec((BR, DIM), lambda i, j: (i, 0)),
            pl.BlockSpec((DIM, BC), lambda i, j: (0, j)),
            pl.BlockSpec((BR, 1), lambda i, j: (i, 0)),
            pl.BlockSpec((1, BC), lambda i, j: (0, j)),
        ],
        out_specs=pl.BlockSpec((BR, K), lambda i, j: (i, 0)),
        out_shape=jax.ShapeDtypeStruct((N, K), jnp.int32),
        scratch_shapes=[
            pltpu.VMEM((BR, BC), jnp.float32),
            pltpu.VMEM((BR, UW), jnp.float32),
            pltpu.VMEM((BR, UW), jnp.int32),
            pltpu.VMEM((BR, TL), jnp.int32),
            pltpu.VMEM((BR, PAD), jnp.float32),
            pltpu.VMEM((BR, PAD), jnp.int32),
            pltpu.VMEM((BR, PAD), jnp.float32),
            pltpu.VMEM((BR, PAD), jnp.int32),
        ],
    )(x, xt, x2r, x2c)
    src = idx.reshape(-1).astype(jnp.int64)
    dst = jnp.repeat(jnp.arange(N, dtype=jnp.int64), K)
    return src, dst


# speedup vs baseline: 1.0078x; 1.0078x over previous
import jax
import jax.numpy as jnp
from jax import lax
from jax.experimental import pallas as pl
from jax.experimental.pallas import tpu as pltpu

N = 8192
DIM = 512
K = 16
BR = 1024
BC = 2048
TL = 128
NT = BC // TL
PAD = 128
RS = 64
UW = 4 * TL
NR = N // BR
NC = N // BC
INF = float("inf")
BIG = 2**30


def _norms_body(x_ref, out_ref):
    x = x_ref[...]
    out_ref[...] = jnp.sum(x * x, axis=1, keepdims=True)


def _cminmax(av, ai, bv, bi):
    c = av <= bv
    return (jnp.where(c, av, bv), jnp.where(c, ai, bi),
            jnp.where(c, bv, av), jnp.where(c, bi, ai))


def _merge11(a, b):
    (av, ai, acnt), (bv, bi, bcnt) = a, b
    lov, loi, hiv, hii = _cminmax(av[0], ai[0], bv[0], bi[0])
    return ([lov, hiv], [loi, hii], acnt + bcnt)


def _merge22(a, b):
    (av, ai, acnt), (bv, bi, bcnt) = a, b
    o0v, o0i, hv, hi = _cminmax(av[0], ai[0], bv[0], bi[0])
    c1 = av[1] <= bv[1]
    m1v = jnp.where(c1, av[1], bv[1])
    m1i = jnp.where(c1, ai[1], bi[1])
    o1v, o1i, o2v, o2i = _cminmax(hv, hi, m1v, m1i)
    return ([o0v, o1v, o2v], [o0i, o1i, o2i], acnt + bcnt)


def _merge33(a, b):
    (av, ai, acnt), (bv, bi, bcnt) = a, b
    o0v, o0i, hv, hi = _cminmax(av[0], ai[0], bv[0], bi[0])
    c1 = av[1] <= bv[1]
    m1v = jnp.where(c1, av[1], bv[1])
    m1i = jnp.where(c1, ai[1], bi[1])
    o1v, o1i, h2v, h2i = _cminmax(hv, hi, m1v, m1i)
    c3 = av[2] <= bv[2]
    m2v = jnp.where(c3, av[2], bv[2])
    m2i = jnp.where(c3, ai[2], bi[2])
    c4 = h2v <= m2v
    o2v = jnp.where(c4, h2v, m2v)
    o2i = jnp.where(c4, h2i, m2i)
    return ([o0v, o1v, o2v], [o0i, o1i, o2i], acnt + bcnt)


def _knn_body(xi_ref, xjt_ref, x2r_ref, x2c_ref, out_ref,
              d_ref, u_ref, ui_ref, cnt_ref, bv_ref, bi_ref,
              nv_ref, ni_ref):
    j = pl.program_id(1)

    @pl.when(j == 0)
    def _init():
        bv_ref[...] = jnp.full((BR, PAD), INF, jnp.float32)
        bi_ref[...] = jnp.full((BR, PAD), BIG, jnp.int32)
        nv_ref[...] = jnp.full((BR, PAD), INF, jnp.float32)
        ni_ref[...] = jnp.full((BR, PAD), BIG, jnp.int32)

    mm = jnp.dot(xi_ref[...], xjt_ref[...],
                 preferred_element_type=jnp.float32)
    d_ref[...] = (x2r_ref[...] + x2c_ref[...]) - 2.0 * mm

    joff = j * BC
    lanes = lax.broadcasted_iota(jnp.int32, (BR, PAD), 1)

    def strip_fold(si, carry):
        r = si * RS
        tv = bv_ref[pl.ds(r, RS), K - 1:K]
        lanes8 = lax.broadcasted_iota(jnp.int32, (RS, TL), 1)
        leaves = []
        for k in range(NT):
            v = d_ref[pl.ds(r, RS), k * TL:(k + 1) * TL]
            keep = v < tv
            fv = jnp.where(keep, v, INF)
            fi = lanes8 + (joff + k * TL)
            leaves.append(([fv], [fi], keep.astype(jnp.int32)))
        leaves = [_merge11(leaves[s], leaves[s + 8]) for s in range(8)]
        leaves = [_merge22(leaves[s], leaves[s + 4]) for s in range(4)]
        leaves = [_merge33(leaves[s], leaves[s + 2]) for s in range(2)]
        s_v, s_i, cnt = _merge33(leaves[0], leaves[1])
        u_ref[pl.ds(r, RS), 0 * TL:1 * TL] = s_v[0]
        u_ref[pl.ds(r, RS), 1 * TL:2 * TL] = s_v[1]
        u_ref[pl.ds(r, RS), 2 * TL:3 * TL] = s_v[2]
        ui_ref[pl.ds(r, RS), 0 * TL:1 * TL] = s_i[0]
        ui_ref[pl.ds(r, RS), 1 * TL:2 * TL] = s_i[1]
        ui_ref[pl.ds(r, RS), 2 * TL:3 * TL] = s_i[2]
        cnt_ref[pl.ds(r, RS), :] = cnt
        return carry

    lax.fori_loop(0, BR // RS, strip_fold, 0)
    overflow = jnp.max(cnt_ref[...]) > 3

    @pl.when(jnp.logical_not(overflow))
    def _fast():
        u_ref[:, 3 * TL:4 * TL] = bv_ref[...]
        ui_ref[:, 3 * TL:4 * TL] = bi_ref[...]

        def pass_u(t, carry):
            uv = u_ref[...]
            uiv = ui_ref[...]
            m = jnp.min(uv, axis=1, keepdims=True)
            am = jnp.min(jnp.where(uv == m, uiv, BIG),
                         axis=1, keepdims=True)
            u_ref[...] = jnp.where(uiv == am, INF, uv)
            nv_ref[...] = jnp.where(lanes == t, m, nv_ref[...])
            ni_ref[...] = jnp.where(lanes == t, am, ni_ref[...])
            return carry

        lax.fori_loop(0, K, pass_u, 0)

    @pl.when(overflow)
    def _slow():
        def pass_t(t, carry):
            dvv = d_ref[...]
            bvv = bv_ref[...]
            biv = bi_ref[...]
            iota = lax.broadcasted_iota(jnp.int32, (BR, BC), 1)
            m = jnp.minimum(jnp.min(dvv, axis=1, keepdims=True),
                            jnp.min(bvv, axis=1, keepdims=True))
            amd = jnp.min(jnp.where(dvv == m, iota, BIG),
                          axis=1, keepdims=True) + joff
            amp = jnp.min(jnp.where(bvv == m, biv, BIG),
                          axis=1, keepdims=True)
            am = jnp.minimum(amp, amd)
            d_ref[...] = jnp.where(iota == (am - joff), INF, dvv)
            bv_ref[...] = jnp.where(biv == am, INF, bvv)
            nv_ref[...] = jnp.where(lanes == t, m, nv_ref[...])
            ni_ref[...] = jnp.where(lanes == t, am, ni_ref[...])
            return carry

        lax.fori_loop(0, K, pass_t, 0)

    bv_ref[...] = nv_ref[...]
    bi_ref[...] = ni_ref[...]
    nv_ref[...] = jnp.full((BR, PAD), INF, jnp.float32)
    ni_ref[...] = jnp.full((BR, PAD), BIG, jnp.int32)

    @pl.when(j == NC - 1)
    def _emit():
        out_ref[...] = bi_ref[:, :K]


def kernel(x):
    x2r = pl.pallas_call(
        _norms_body,
        out_shape=jax.ShapeDtypeStruct((N, 1), jnp.float32),
    )(x)
    xt = x.T
    x2c = x2r.T
    idx = pl.pallas_call(
        _knn_body,
        grid=(NR, NC),
        in_specs=[
            pl.BlockSpec((BR, DIM), lambda i, j: (i, 0)),
            pl.BlockSpec((DIM, BC), lambda i, j: (0, j)),
            pl.BlockSpec((BR, 1), lambda i, j: (i, 0)),
            pl.BlockSpec((1, BC), lambda i, j: (0, j)),
        ],
        out_specs=pl.BlockSpec((BR, K), lambda i, j: (i, 0)),
        out_shape=jax.ShapeDtypeStruct((N, K), jnp.int32),
        scratch_shapes=[
            pltpu.VMEM((BR, BC), jnp.float32),
            pltpu.VMEM((BR, UW), jnp.float32),
            pltpu.VMEM((BR, UW), jnp.int32),
            pltpu.VMEM((BR, TL), jnp.int32),
            pltpu.VMEM((BR, PAD), jnp.float32),
            pltpu.VMEM((BR, PAD), jnp.int32),
            pltpu.VMEM((BR, PAD), jnp.float32),
            pltpu.VMEM((BR, PAD), jnp.int32),
        ],
    )(x, xt, x2r, x2c)
    src = idx.reshape(-1).astype(jnp.int64)
    dst = jnp.repeat(jnp.arange(N, dtype=jnp.int64), K)
    return src, dst
